# baseline (speedup 1.0000x reference)
import functools

import jax
import jax.numpy as jnp
from jax.experimental import pallas as pl
from jax.experimental.pallas import tpu as pltpu

_PREC = jax.lax.Precision.DEFAULT


def _dot(a, b):
    return jax.lax.dot_general(
        a, b, (((1,), (0,)), ((), ())),
        precision=_PREC, preferred_element_type=jnp.float32)


def _xw_kernel(x_ref, w_ref, o_ref):
    o_ref[:] = _dot(jnp.maximum(x_ref[:], 0.0), w_ref[:])


def _xw_pass(x, w, bm=1024):
    m, k = x.shape
    c = w.shape[1]
    return pl.pallas_call(
        _xw_kernel,
        grid=(m // bm,),
        in_specs=[
            pl.BlockSpec((bm, k), lambda i: (i, 0)),
            pl.BlockSpec((k, c), lambda i: (0, 0)),
        ],
        out_specs=pl.BlockSpec((bm, c), lambda i: (i, 0)),
        out_shape=jax.ShapeDtypeStruct((m, c), jnp.float32),
        compiler_params=pltpu.CompilerParams(
            dimension_semantics=("parallel",)),
    )(x, w)


def _stream_kernel(n_ref, a_ref, o_ref):
    o_ref[:] = jnp.maximum(_dot(n_ref[:], a_ref[:]), 0.0)


def _head_kernel(x0b_ref, x1_ref, x2_ref,
                 w0_ref, b0_ref, w1_ref, b1_ref, w2_ref, b2_ref, o_ref):
    m0 = jnp.sum(x0b_ref[:], axis=0, keepdims=True) / x0b_ref.shape[0]
    m1 = (jnp.sum(jnp.maximum(x1_ref[:], 0.0), axis=0, keepdims=True)
          / x1_ref.shape[0])
    m2 = jnp.sum(x2_ref[:], axis=0, keepdims=True) / x2_ref.shape[0]
    o_ref[:] = (_dot(m0, w0_ref[:]) + b0_ref[:]
                + _dot(m1, w1_ref[:]) + b1_ref[:]
                + _dot(m2, w2_ref[:]) + b2_ref[:])


def _stream_pass(n, a, bm):
    m, k = n.shape
    c = a.shape[1]
    grid = (m // bm,)
    return pl.pallas_call(
        _stream_kernel,
        grid=grid,
        in_specs=[
            pl.BlockSpec((bm, k), lambda i: (i, 0)),
            pl.BlockSpec((k, c), lambda i: (0, 0)),
        ],
        out_specs=pl.BlockSpec((bm, c), lambda i: (i, 0)),
        out_shape=jax.ShapeDtypeStruct((m, c), jnp.float32),
        compiler_params=pltpu.CompilerParams(
            dimension_semantics=("parallel",)),
    )(n, a)


def kernel(x_0, x_1, neighborhood_0_to_0, neighborhood_1_to_2,
           w00_l0, w12_l0, w00_l1, w12_l1,
           lin0_w, lin0_b, lin1_w, lin1_b, lin2_w, lin2_b):
    n_nodes, c0 = x_0.shape
    n_edges, c1 = x_1.shape
    n_faces = neighborhood_1_to_2.shape[0]
    c2 = w12_l1.shape[1]
    ncls = lin0_w.shape[1]

    a0 = _xw_pass(x_0, w00_l0)
    b = _xw_pass(x_1, w12_l1)

    x0a = _stream_pass(neighborhood_0_to_0, a0, bm=512)

    a1 = _xw_pass(x0a, w00_l1)

    x0b = _stream_pass(neighborhood_0_to_0, a1, bm=512)

    x2 = _stream_pass(neighborhood_1_to_2, b, bm=256)

    out = pl.pallas_call(
        _head_kernel,
        out_shape=jax.ShapeDtypeStruct((1, ncls), jnp.float32),
    )(x0b, x_1, x2,
      lin0_w, lin0_b.reshape(1, ncls), lin1_w, lin1_b.reshape(1, ncls),
      lin2_w, lin2_b.reshape(1, ncls))
    return out.reshape(ncls)


# speedup vs baseline: 1.0146x; 1.0146x over previous
import functools

import jax
import jax.numpy as jnp
from jax.experimental import pallas as pl
from jax.experimental.pallas import tpu as pltpu

_PREC = jax.lax.Precision.DEFAULT


def _dot(a, b):
    return jax.lax.dot_general(
        a, b, (((1,), (0,)), ((), ())),
        precision=_PREC, preferred_element_type=jnp.float32)


def _xw_kernel(x_ref, w_ref, o_ref):
    o_ref[:] = _dot(jnp.maximum(x_ref[:], 0.0), w_ref[:])


def _xw_pass(x, w, bm=1024):
    m, k = x.shape
    c = w.shape[1]
    return pl.pallas_call(
        _xw_kernel,
        grid=(m // bm,),
        in_specs=[
            pl.BlockSpec((bm, k), lambda i: (i, 0)),
            pl.BlockSpec((k, c), lambda i: (0, 0)),
        ],
        out_specs=pl.BlockSpec((bm, c), lambda i: (i, 0)),
        out_shape=jax.ShapeDtypeStruct((m, c), jnp.float32),
        compiler_params=pltpu.CompilerParams(
            dimension_semantics=("parallel",)),
    )(x, w)


def _stream_kernel(n_ref, a_ref, o_ref):
    o_ref[:] = jnp.maximum(
        jax.lax.dot_general(
            n_ref[:].astype(jnp.bfloat16), a_ref[:].astype(jnp.bfloat16),
            (((1,), (0,)), ((), ())),
            preferred_element_type=jnp.float32), 0.0)


def _head_kernel(x0b_ref, x1_ref, x2_ref,
                 w0_ref, b0_ref, w1_ref, b1_ref, w2_ref, b2_ref, o_ref):
    m0 = jnp.sum(x0b_ref[:], axis=0, keepdims=True) / x0b_ref.shape[0]
    m1 = (jnp.sum(jnp.maximum(x1_ref[:], 0.0), axis=0, keepdims=True)
          / x1_ref.shape[0])
    m2 = jnp.sum(x2_ref[:], axis=0, keepdims=True) / x2_ref.shape[0]
    o_ref[:] = (_dot(m0, w0_ref[:]) + b0_ref[:]
                + _dot(m1, w1_ref[:]) + b1_ref[:]
                + _dot(m2, w2_ref[:]) + b2_ref[:])


def _stream_pass(n, a, bm):
    m, k = n.shape
    c = a.shape[1]
    grid = (m // bm,)
    return pl.pallas_call(
        _stream_kernel,
        grid=grid,
        in_specs=[
            pl.BlockSpec((bm, k), lambda i: (i, 0)),
            pl.BlockSpec((k, c), lambda i: (0, 0)),
        ],
        out_specs=pl.BlockSpec((bm, c), lambda i: (i, 0)),
        out_shape=jax.ShapeDtypeStruct((m, c), jnp.float32),
        compiler_params=pltpu.CompilerParams(
            dimension_semantics=("parallel",)),
    )(n, a)


def kernel(x_0, x_1, neighborhood_0_to_0, neighborhood_1_to_2,
           w00_l0, w12_l0, w00_l1, w12_l1,
           lin0_w, lin0_b, lin1_w, lin1_b, lin2_w, lin2_b):
    n_nodes, c0 = x_0.shape
    n_edges, c1 = x_1.shape
    n_faces = neighborhood_1_to_2.shape[0]
    c2 = w12_l1.shape[1]
    ncls = lin0_w.shape[1]

    a0 = _xw_pass(x_0, w00_l0)
    b = _xw_pass(x_1, w12_l1)

    x0a = _stream_pass(neighborhood_0_to_0, a0, bm=256)

    a1 = _xw_pass(x0a, w00_l1)

    x0b = _stream_pass(neighborhood_0_to_0, a1, bm=256)

    x2 = _stream_pass(neighborhood_1_to_2, b, bm=256)

    out = pl.pallas_call(
        _head_kernel,
        out_shape=jax.ShapeDtypeStruct((1, ncls), jnp.float32),
    )(x0b, x_1, x2,
      lin0_w, lin0_b.reshape(1, ncls), lin1_w, lin1_b.reshape(1, ncls),
      lin2_w, lin2_b.reshape(1, ncls))
    return out.reshape(ncls)


# speedup vs baseline: 1.0908x; 1.0751x over previous
import jax
import jax.numpy as jnp
from jax.experimental import pallas as pl
from jax.experimental.pallas import tpu as pltpu


def _dot_f32(a, b):
    return jax.lax.dot_general(
        a, b, (((1,), (0,)), ((), ())),
        precision=jax.lax.Precision.DEFAULT,
        preferred_element_type=jnp.float32)


def _xwt_kernel(x_ref, w_ref, o_ref):
    o_ref[:] = jax.lax.dot_general(
        w_ref[:], jnp.maximum(x_ref[:], 0.0),
        (((0,), (1,)), ((), ())),
        precision=jax.lax.Precision.DEFAULT,
        preferred_element_type=jnp.float32)


def _xw_t_pass(x, w, bm=2048):
    m, k = x.shape
    c = w.shape[1]
    return pl.pallas_call(
        _xwt_kernel,
        grid=(m // bm,),
        in_specs=[
            pl.BlockSpec((bm, k), lambda i: (i, 0)),
            pl.BlockSpec((k, c), lambda i: (0, 0)),
        ],
        out_specs=pl.BlockSpec((c, bm), lambda i: (0, i)),
        out_shape=jax.ShapeDtypeStruct((c, m), jnp.float32),
        compiler_params=pltpu.CompilerParams(
            dimension_semantics=("parallel",)),
    )(x, w)


def _stream_kernel(n_ref, at_ref, o_ref):
    o_ref[:] = jnp.maximum(
        jax.lax.dot_general(
            at_ref[:], n_ref[:],
            (((1,), (1,)), ((), ())),
            precision=jax.lax.Precision.DEFAULT,
            preferred_element_type=jnp.float32), 0.0)


def _stream_t_pass(n, at, bm):
    m, k = n.shape
    c = at.shape[0]
    return pl.pallas_call(
        _stream_kernel,
        grid=(m // bm,),
        in_specs=[
            pl.BlockSpec((bm, k), lambda i: (i, 0)),
            pl.BlockSpec((c, k), lambda i: (0, 0)),
        ],
        out_specs=pl.BlockSpec((c, bm), lambda i: (0, i)),
        out_shape=jax.ShapeDtypeStruct((c, m), jnp.float32),
        compiler_params=pltpu.CompilerParams(
            dimension_semantics=("parallel",)),
    )(n, at)


def _bf16_kernel(x_ref, o_ref):
    o_ref[:] = x_ref[:].astype(jnp.bfloat16)


def _to_bf16(x):
    return pl.pallas_call(
        _bf16_kernel,
        out_shape=jax.ShapeDtypeStruct(x.shape, jnp.bfloat16),
    )(x)


def _mid_t_kernel(x0at_ref, w_ref, o_ref):
    o_ref[:] = jax.lax.dot_general(
        w_ref[:], x0at_ref[:], (((0,), (0,)), ((), ())),
        precision=jax.lax.Precision.DEFAULT,
        preferred_element_type=jnp.float32)


def _head_kernel(x0bt_ref, x1_ref, x2t_ref,
                 w0_ref, b0_ref, w1_ref, b1_ref, w2_ref, b2_ref, o_ref):
    m0 = jnp.sum(x0bt_ref[:], axis=1, keepdims=True) / x0bt_ref.shape[1]
    m1 = (jnp.sum(jnp.maximum(x1_ref[:], 0.0), axis=0, keepdims=True)
          / x1_ref.shape[0])
    m2 = jnp.sum(x2t_ref[:], axis=1, keepdims=True) / x2t_ref.shape[1]
    o_ref[:] = (
        jax.lax.dot_general(m0, w0_ref[:], (((0,), (0,)), ((), ())),
                            preferred_element_type=jnp.float32)
        + b0_ref[:]
        + _dot_f32(m1, w1_ref[:]) + b1_ref[:]
        + jax.lax.dot_general(m2, w2_ref[:], (((0,), (0,)), ((), ())),
                              preferred_element_type=jnp.float32)
        + b2_ref[:])


def kernel(x_0, x_1, neighborhood_0_to_0, neighborhood_1_to_2,
           w00_l0, w12_l0, w00_l1, w12_l1,
           lin0_w, lin0_b, lin1_w, lin1_b, lin2_w, lin2_b):
    ncls = lin0_w.shape[1]

    a0t = _xw_t_pass(x_0, w00_l0)
    bt = _xw_t_pass(x_1, w12_l1)

    x0at = _stream_t_pass(neighborhood_0_to_0, a0t, bm=256)

    a1t = pl.pallas_call(
        _mid_t_kernel,
        out_shape=jax.ShapeDtypeStruct(x0at.shape, jnp.float32),
    )(x0at, w00_l1)

    x0bt = _stream_t_pass(neighborhood_0_to_0, a1t, bm=256)

    x2t = _stream_t_pass(neighborhood_1_to_2, bt, bm=256)

    out = pl.pallas_call(
        _head_kernel,
        out_shape=jax.ShapeDtypeStruct((1, ncls), jnp.float32),
    )(x0bt, x_1, x2t,
      lin0_w, lin0_b.reshape(1, ncls), lin1_w, lin1_b.reshape(1, ncls),
      lin2_w, lin2_b.reshape(1, ncls))
    return out.reshape(ncls)
